# baseline (speedup 1.0000x reference)
import functools

import jax
import jax.numpy as jnp
from jax import lax
from jax.experimental import pallas as pl
from jax.experimental.pallas import tpu as pltpu
from jax.experimental.pallas import tpu_sc as plsc

N_NODES = 10000
N_EDGES = 160000
D = 256
DH = 128
NC, NS, L = 2, 16, 16

EDGES_PER_TILE = N_EDGES // NS
CHUNK = 112
N_CHUNKS = 90
EDGES_PER_TILE_P = N_CHUNKS * CHUNK
NBUF = 3
DCHUNK = 100
NHALF = 2
HCHUNKS = (EDGES_PER_TILE // DCHUNK) // NHALF
NPAD = 10240
ROWS_PER_TILE = NPAD // NS
ZROWS = 80

HIST_PAD = 10240
HROWS = HIST_PAD // NS


def _zero_vec_ref(ref, n):
    def body(i, _):
        ref[pl.ds(i * 16, 16)] = jnp.zeros((16,), jnp.float32)
        return 0
    lax.fori_loop(0, n // 16, body, 0)


def _zero_mat_ref(ref, rows, cols):
    def body(i, _):
        def inner(j, _):
            ref[i, pl.ds(j * 16, 16)] = jnp.zeros((16,), jnp.float32)
            return 0
        lax.fori_loop(0, cols // 16, inner, 0)
        return 0
    lax.fori_loop(0, rows, body, 0)


def _sc_degrees_body(src_hbm, dst_hbm, dego_hbm, degi_hbm,
                     idx_all, ones_v, bounce_v, acc_sh):
    c = lax.axis_index("c")
    s = lax.axis_index("s")

    _zero_vec_ref(bounce_v, HROWS)

    def ones_body(i, _):
        ones_v[pl.ds(i * 16, 16)] = jnp.ones((16,), jnp.float32)
        return 0
    lax.fori_loop(0, 8, ones_body, 0)

    pltpu.sync_copy(bounce_v, acc_sh.at[pl.ds(s * HROWS, HROWS)])
    plsc.subcore_barrier()

    def run(edge_hbm):
        for hh in range(NHALF):
            pltpu.sync_copy(edge_hbm.at[s, hh], idx_all)

            def body(i, _):
                pltpu.sync_copy(ones_v.at[pl.ds(0, DCHUNK)],
                                acc_sh.at[idx_all.at[i]], add=True)
                return 0
            lax.fori_loop(0, HCHUNKS, body, 0)

    @pl.when(c == 0)
    def _():
        run(src_hbm)

    @pl.when(c == 1)
    def _():
        run(dst_hbm)

    plsc.subcore_barrier()

    @pl.when(c == 0)
    def _():
        pltpu.sync_copy(acc_sh.at[pl.ds(s * HROWS, HROWS)],
                        dego_hbm.at[pl.ds(s * HROWS, HROWS)])

    @pl.when(c == 1)
    def _():
        pltpu.sync_copy(acc_sh.at[pl.ds(s * HROWS, HROWS)],
                        degi_hbm.at[pl.ds(s * HROWS, HROWS)])


_sc_degrees = pl.kernel(
    _sc_degrees_body,
    out_type=[jax.ShapeDtypeStruct((HIST_PAD,), jnp.float32),
              jax.ShapeDtypeStruct((HIST_PAD,), jnp.float32)],
    mesh=plsc.VectorSubcoreMesh(core_axis_name="c", subcore_axis_name="s"),
    scratch_types=[
        pltpu.VMEM((HCHUNKS, DCHUNK), jnp.int32),
        pltpu.VMEM((128,), jnp.float32),
        pltpu.VMEM((HROWS,), jnp.float32),
        pltpu.VMEM_SHARED((HIST_PAD,), jnp.float32),
    ],
)


def _sc_agg_body(g0_hbm, g1_hbm, src_hbm, dst_hbm, out0_hbm, out1_hbm,
                 sidx0, sidx1, sidx2, didx0, didx1, didx2,
                 rows, acc_sh, zsem,
                 gsem0, gsem1, gsem2, isem0, isem1, isem2):
    c = lax.axis_index("c")
    s = lax.axis_index("s")
    sidx = (sidx0, sidx1, sidx2)
    didx = (didx0, didx1, didx2)
    gsems = (gsem0, gsem1, gsem2)
    isems = (isem0, isem1, isem2)

    zbuf = rows.at[0]
    _zero_mat_ref(zbuf, ZROWS, DH)
    for k in range(ROWS_PER_TILE // ZROWS):
        pltpu.async_copy(zbuf.at[pl.ds(0, ZROWS)],
                         acc_sh.at[pl.ds(s * ROWS_PER_TILE + k * ZROWS, ZROWS)], zsem)
    for k in range(NBUF - 1):
        pltpu.sync_copy(src_hbm.at[s, k], sidx[k])
        pltpu.sync_copy(dst_hbm.at[s, k], didx[k])
    for k in range(ROWS_PER_TILE // ZROWS):
        pltpu.make_async_copy(zbuf.at[pl.ds(0, ZROWS)],
                              acc_sh.at[pl.ds(s * ROWS_PER_TILE + k * ZROWS, ZROWS)], zsem).wait()
    plsc.subcore_barrier()

    def run(g_hbm):
        for k in range(NBUF - 1):
            pltpu.async_copy(g_hbm.at[sidx[k].at[0]], rows.at[k], gsems[k])
        pltpu.async_copy(src_hbm.at[s, NBUF - 1], sidx[NBUF - 1], isems[NBUF - 1])
        pltpu.async_copy(dst_hbm.at[s, NBUF - 1], didx[NBUF - 1], isems[NBUF - 1])

        def visit(cc, k):
            kn = (k + NBUF - 1) % NBUF
            pltpu.make_async_copy(g_hbm.at[sidx[k].at[0]], rows.at[k], gsems[k]).wait()

            @pl.when(cc + NBUF - 1 < N_CHUNKS)
            def _():
                pltpu.make_async_copy(src_hbm.at[s, cc + NBUF - 1], sidx[kn], isems[kn]).wait()
                pltpu.make_async_copy(dst_hbm.at[s, cc + NBUF - 1], didx[kn], isems[kn]).wait()
                pltpu.async_copy(g_hbm.at[sidx[kn].at[0]], rows.at[kn], gsems[kn])

            pltpu.sync_copy(rows.at[k], acc_sh.at[didx[k].at[0]], add=True)

            @pl.when(cc + NBUF < N_CHUNKS)
            def _():
                pltpu.async_copy(src_hbm.at[s, cc + NBUF], sidx[k], isems[k])
                pltpu.async_copy(dst_hbm.at[s, cc + NBUF], didx[k], isems[k])

        def body(j, _):
            for k in range(NBUF):
                cc = NBUF * j + k

                @pl.when(cc < N_CHUNKS)
                def _():
                    visit(cc, k)
            return 0
        lax.fori_loop(0, (N_CHUNKS + NBUF - 1) // NBUF, body, 0)

    @pl.when(c == 0)
    def _():
        run(g0_hbm)

    @pl.when(c == 1)
    def _():
        run(g1_hbm)

    plsc.subcore_barrier()

    def writeback(out_hbm):
        r0 = s * ROWS_PER_TILE
        pltpu.sync_copy(acc_sh.at[pl.ds(r0, ROWS_PER_TILE)],
                        out_hbm.at[pl.ds(r0, ROWS_PER_TILE)])

    @pl.when(c == 0)
    def _():
        writeback(out0_hbm)

    @pl.when(c == 1)
    def _():
        writeback(out1_hbm)


_sc_agg = pl.kernel(
    _sc_agg_body,
    out_type=[jax.ShapeDtypeStruct((NPAD, DH), jnp.float32),
              jax.ShapeDtypeStruct((NPAD, DH), jnp.float32)],
    mesh=plsc.VectorSubcoreMesh(core_axis_name="c", subcore_axis_name="s"),
    scratch_types=(
        [pltpu.VMEM((1, CHUNK), jnp.int32) for _ in range(2 * NBUF)]
        + [
            pltpu.VMEM((NBUF, CHUNK, DH), jnp.float32),
            pltpu.VMEM_SHARED((NPAD, DH), jnp.float32),
        ]
        + [pltpu.SemaphoreType.DMA for _ in range(1 + 2 * NBUF)]
    ),
)


_RB = 1000


def _prelude_body(dego_ref, degi_ref, x_ref, ns_ref, nd_ref, g0_ref, g1_ref):
    ns = lax.rsqrt(jnp.maximum(dego_ref[...], 1.0))
    nd = lax.rsqrt(jnp.maximum(degi_ref[...], 1.0))
    ns_ref[...] = ns
    nd_ref[...] = nd
    g = x_ref[...] * ns
    g0_ref[...] = g[:, :DH]
    g1_ref[...] = g[:, DH:]


_prelude = pl.pallas_call(
    _prelude_body,
    grid=(N_NODES // _RB,),
    in_specs=[
        pl.BlockSpec((_RB, 1), lambda i: (i, 0)),
        pl.BlockSpec((_RB, 1), lambda i: (i, 0)),
        pl.BlockSpec((_RB, D), lambda i: (i, 0)),
    ],
    out_specs=[
        pl.BlockSpec((_RB, 1), lambda i: (i, 0)),
        pl.BlockSpec((_RB, 1), lambda i: (i, 0)),
        pl.BlockSpec((_RB, DH), lambda i: (i, 0)),
        pl.BlockSpec((_RB, DH), lambda i: (i, 0)),
    ],
    out_shape=[
        jax.ShapeDtypeStruct((N_NODES, 1), jnp.float32),
        jax.ShapeDtypeStruct((N_NODES, 1), jnp.float32),
        jax.ShapeDtypeStruct((N_NODES, DH), jnp.float32),
        jax.ShapeDtypeStruct((N_NODES, DH), jnp.float32),
    ],
)


def _layer_body(a0_ref, a1_ref, nd_ref, ns_ref, w_ref, b_ref, *out_refs,
                emit_g):
    nd = nd_ref[...]
    agg = jnp.concatenate([a0_ref[...] * nd, a1_ref[...] * nd], axis=1)
    z = jnp.dot(agg, w_ref[...], preferred_element_type=jnp.float32,
                precision=lax.Precision.HIGHEST) + b_ref[...]
    h = jnp.maximum(z, 0.0)
    if emit_g:
        g0_ref, g1_ref = out_refs
        g = h * ns_ref[...]
        g0_ref[...] = g[:, :DH]
        g1_ref[...] = g[:, DH:]
    else:
        (h_ref,) = out_refs
        h_ref[...] = h


def _make_layer(emit_g):
    if emit_g:
        out_specs = [pl.BlockSpec((_RB, DH), lambda i: (i, 0)),
                     pl.BlockSpec((_RB, DH), lambda i: (i, 0))]
        out_shape = [jax.ShapeDtypeStruct((N_NODES, DH), jnp.float32),
                     jax.ShapeDtypeStruct((N_NODES, DH), jnp.float32)]
    else:
        out_specs = [pl.BlockSpec((_RB, D), lambda i: (i, 0))]
        out_shape = [jax.ShapeDtypeStruct((N_NODES, D), jnp.float32)]
    return pl.pallas_call(
        functools.partial(_layer_body, emit_g=emit_g),
        grid=(N_NODES // _RB,),
        in_specs=[
            pl.BlockSpec((_RB, DH), lambda i: (i, 0)),
            pl.BlockSpec((_RB, DH), lambda i: (i, 0)),
            pl.BlockSpec((_RB, 1), lambda i: (i, 0)),
            pl.BlockSpec((_RB, 1), lambda i: (i, 0)),
            pl.BlockSpec((D, D), lambda i: (0, 0)),
            pl.BlockSpec((1, D), lambda i: (0, 0)),
        ],
        out_specs=out_specs,
        out_shape=out_shape,
    )


_layer_mid = _make_layer(emit_g=True)
_layer_last = _make_layer(emit_g=False)


def kernel(x, edge_index, W0, b0, W1, b1, W2, b2):
    src_d = edge_index[0].reshape(NS, NHALF, HCHUNKS, DCHUNK)
    dst_d = edge_index[1].reshape(NS, NHALF, HCHUNKS, DCHUNK)
    pad_n = NS * EDGES_PER_TILE_P - N_EDGES
    src = jnp.concatenate(
        [edge_index[0], jnp.zeros((pad_n,), jnp.int32)]
    ).reshape(NS, N_CHUNKS, 1, CHUNK)
    dst = jnp.concatenate(
        [edge_index[1], jnp.full((pad_n,), N_NODES, jnp.int32)]
    ).reshape(NS, N_CHUNKS, 1, CHUNK)

    dego_p, degi_p = _sc_degrees(src_d, dst_d)
    dego = dego_p[:N_NODES].reshape(N_NODES, 1)
    degi = degi_p[:N_NODES].reshape(N_NODES, 1)

    ns, nd, g0, g1 = _prelude(dego, degi, x)

    a0, a1 = _sc_agg(g0, g1, src, dst)
    g0, g1 = _layer_mid(a0[:N_NODES], a1[:N_NODES], nd, ns, W0, b0.reshape(1, D))

    a0, a1 = _sc_agg(g0, g1, src, dst)
    g0, g1 = _layer_mid(a0[:N_NODES], a1[:N_NODES], nd, ns, W1, b1.reshape(1, D))

    a0, a1 = _sc_agg(g0, g1, src, dst)
    (h,) = _layer_last(a0[:N_NODES], a1[:N_NODES], nd, ns, W2, b2.reshape(1, D))
    return h


# speedup vs baseline: 1.3227x; 1.3227x over previous
import functools

import jax
import jax.numpy as jnp
from jax import lax
from jax.experimental import pallas as pl
from jax.experimental.pallas import tpu as pltpu
from jax.experimental.pallas import tpu_sc as plsc

N_NODES = 10000
N_EDGES = 160000
D = 256
DH = 128
NC, NS, L = 2, 16, 16

EDGES_PER_TILE = N_EDGES // NS
CHUNK = 100
N_CHUNKS = EDGES_PER_TILE // CHUNK
NBUF = 3
DCHUNK = 100
NHALF = 2
HCHUNKS = (EDGES_PER_TILE // DCHUNK) // NHALF
NPAD = 10240
ROWS_PER_TILE = NPAD // NS
ZROWS = 80

HIST_PAD = 10240
HROWS = HIST_PAD // NS


def _zero_vec_ref(ref, n):
    def body(i, _):
        ref[pl.ds(i * 16, 16)] = jnp.zeros((16,), jnp.float32)
        return 0
    lax.fori_loop(0, n // 16, body, 0)


def _zero_mat_ref(ref, rows, cols):
    def body(i, _):
        def inner(j, _):
            ref[i, pl.ds(j * 16, 16)] = jnp.zeros((16,), jnp.float32)
            return 0
        lax.fori_loop(0, cols // 16, inner, 0)
        return 0
    lax.fori_loop(0, rows, body, 0)


def _sc_degrees_body(src_hbm, dst_hbm, dego_hbm, degi_hbm,
                     idx_all, ones_v, bounce_v, acc_sh):
    c = lax.axis_index("c")
    s = lax.axis_index("s")

    _zero_vec_ref(bounce_v, HROWS)

    def ones_body(i, _):
        ones_v[pl.ds(i * 16, 16)] = jnp.ones((16,), jnp.float32)
        return 0
    lax.fori_loop(0, 8, ones_body, 0)

    pltpu.sync_copy(bounce_v, acc_sh.at[pl.ds(s * HROWS, HROWS)])
    plsc.subcore_barrier()

    def run(edge_hbm):
        for hh in range(NHALF):
            pltpu.sync_copy(edge_hbm.at[s, hh], idx_all)

            def body(i, _):
                pltpu.sync_copy(ones_v.at[pl.ds(0, DCHUNK)],
                                acc_sh.at[idx_all.at[i]], add=True)
                return 0
            lax.fori_loop(0, HCHUNKS, body, 0)

    @pl.when(c == 0)
    def _():
        run(src_hbm)

    @pl.when(c == 1)
    def _():
        run(dst_hbm)

    plsc.subcore_barrier()

    @pl.when(c == 0)
    def _():
        pltpu.sync_copy(acc_sh.at[pl.ds(s * HROWS, HROWS)],
                        dego_hbm.at[pl.ds(s * HROWS, HROWS)])

    @pl.when(c == 1)
    def _():
        pltpu.sync_copy(acc_sh.at[pl.ds(s * HROWS, HROWS)],
                        degi_hbm.at[pl.ds(s * HROWS, HROWS)])


_sc_degrees = pl.kernel(
    _sc_degrees_body,
    out_type=[jax.ShapeDtypeStruct((HIST_PAD,), jnp.float32),
              jax.ShapeDtypeStruct((HIST_PAD,), jnp.float32)],
    mesh=plsc.VectorSubcoreMesh(core_axis_name="c", subcore_axis_name="s"),
    scratch_types=[
        pltpu.VMEM((HCHUNKS, DCHUNK), jnp.int32),
        pltpu.VMEM((128,), jnp.float32),
        pltpu.VMEM((HROWS,), jnp.float32),
        pltpu.VMEM_SHARED((HIST_PAD,), jnp.float32),
    ],
)


def _sc_agg_body(g0_hbm, g1_hbm, src_hbm, dst_hbm, out0_hbm, out1_hbm,
                 sidx0, sidx1, sidx2, didx0, didx1, didx2,
                 rows, acc_sh, zsem,
                 gsem0, gsem1, gsem2, isem0, isem1, isem2):
    c = lax.axis_index("c")
    s = lax.axis_index("s")
    sidx = (sidx0, sidx1, sidx2)
    didx = (didx0, didx1, didx2)
    gsems = (gsem0, gsem1, gsem2)
    isems = (isem0, isem1, isem2)

    zbuf = rows.at[0]
    _zero_mat_ref(zbuf, ZROWS, DH)
    for k in range(ROWS_PER_TILE // ZROWS):
        pltpu.async_copy(zbuf.at[pl.ds(0, ZROWS)],
                         acc_sh.at[pl.ds(s * ROWS_PER_TILE + k * ZROWS, ZROWS)], zsem)
    for k in range(NBUF - 1):
        pltpu.sync_copy(src_hbm.at[s, k], sidx[k])
        pltpu.sync_copy(dst_hbm.at[s, k], didx[k])
    for k in range(ROWS_PER_TILE // ZROWS):
        pltpu.make_async_copy(zbuf.at[pl.ds(0, ZROWS)],
                              acc_sh.at[pl.ds(s * ROWS_PER_TILE + k * ZROWS, ZROWS)], zsem).wait()
    plsc.subcore_barrier()

    def run(g_hbm):
        for k in range(NBUF - 1):
            pltpu.async_copy(g_hbm.at[sidx[k].at[0]], rows.at[k], gsems[k])
        pltpu.async_copy(src_hbm.at[s, NBUF - 1], sidx[NBUF - 1], isems[NBUF - 1])
        pltpu.async_copy(dst_hbm.at[s, NBUF - 1], didx[NBUF - 1], isems[NBUF - 1])

        def visit(cc, k):
            kn = (k + NBUF - 1) % NBUF
            pltpu.make_async_copy(g_hbm.at[sidx[k].at[0]], rows.at[k], gsems[k]).wait()

            @pl.when(cc + NBUF - 1 < N_CHUNKS)
            def _():
                pltpu.make_async_copy(src_hbm.at[s, cc + NBUF - 1], sidx[kn], isems[kn]).wait()
                pltpu.make_async_copy(dst_hbm.at[s, cc + NBUF - 1], didx[kn], isems[kn]).wait()
                pltpu.async_copy(g_hbm.at[sidx[kn].at[0]], rows.at[kn], gsems[kn])

            pltpu.sync_copy(rows.at[k], acc_sh.at[didx[k].at[0]], add=True)

            @pl.when(cc + NBUF < N_CHUNKS)
            def _():
                pltpu.async_copy(src_hbm.at[s, cc + NBUF], sidx[k], isems[k])
                pltpu.async_copy(dst_hbm.at[s, cc + NBUF], didx[k], isems[k])

        def body(j, _):
            for k in range(NBUF):
                cc = NBUF * j + k

                @pl.when(cc < N_CHUNKS)
                def _():
                    visit(cc, k)
            return 0
        lax.fori_loop(0, (N_CHUNKS + NBUF - 1) // NBUF, body, 0)

    @pl.when(c == 0)
    def _():
        run(g0_hbm)

    @pl.when(c == 1)
    def _():
        run(g1_hbm)

    plsc.subcore_barrier()

    def writeback(out_hbm):
        r0 = s * ROWS_PER_TILE
        pltpu.sync_copy(acc_sh.at[pl.ds(r0, ROWS_PER_TILE)],
                        out_hbm.at[pl.ds(r0, ROWS_PER_TILE)])

    @pl.when(c == 0)
    def _():
        writeback(out0_hbm)

    @pl.when(c == 1)
    def _():
        writeback(out1_hbm)


_sc_agg = pl.kernel(
    _sc_agg_body,
    out_type=[jax.ShapeDtypeStruct((NPAD, DH), jnp.float32),
              jax.ShapeDtypeStruct((NPAD, DH), jnp.float32)],
    mesh=plsc.VectorSubcoreMesh(core_axis_name="c", subcore_axis_name="s"),
    scratch_types=(
        [pltpu.VMEM((1, CHUNK), jnp.int32) for _ in range(2 * NBUF)]
        + [
            pltpu.VMEM((NBUF, CHUNK, DH), jnp.float32),
            pltpu.VMEM_SHARED((NPAD, DH), jnp.float32),
        ]
        + [pltpu.SemaphoreType.DMA for _ in range(1 + 2 * NBUF)]
    ),
)


_RB = 1000


def _prelude_body(dego_ref, degi_ref, x_ref, ns_ref, nd_ref, g0_ref, g1_ref):
    ns = lax.rsqrt(jnp.maximum(dego_ref[...], 1.0))
    nd = lax.rsqrt(jnp.maximum(degi_ref[...], 1.0))
    ns_ref[...] = ns
    nd_ref[...] = nd
    g = x_ref[...] * ns
    g0_ref[...] = g[:, :DH]
    g1_ref[...] = g[:, DH:]


_prelude = pl.pallas_call(
    _prelude_body,
    grid=(N_NODES // _RB,),
    in_specs=[
        pl.BlockSpec((_RB, 1), lambda i: (i, 0)),
        pl.BlockSpec((_RB, 1), lambda i: (i, 0)),
        pl.BlockSpec((_RB, D), lambda i: (i, 0)),
    ],
    out_specs=[
        pl.BlockSpec((_RB, 1), lambda i: (i, 0)),
        pl.BlockSpec((_RB, 1), lambda i: (i, 0)),
        pl.BlockSpec((_RB, DH), lambda i: (i, 0)),
        pl.BlockSpec((_RB, DH), lambda i: (i, 0)),
    ],
    out_shape=[
        jax.ShapeDtypeStruct((N_NODES, 1), jnp.float32),
        jax.ShapeDtypeStruct((N_NODES, 1), jnp.float32),
        jax.ShapeDtypeStruct((N_NODES, DH), jnp.float32),
        jax.ShapeDtypeStruct((N_NODES, DH), jnp.float32),
    ],
)


def _layer_body(a0_ref, a1_ref, nd_ref, ns_ref, w_ref, b_ref, *out_refs,
                emit_g):
    nd = nd_ref[...]
    agg = jnp.concatenate([a0_ref[...] * nd, a1_ref[...] * nd], axis=1)
    z = jnp.dot(agg, w_ref[...], preferred_element_type=jnp.float32,
                precision=lax.Precision.HIGHEST) + b_ref[...]
    h = jnp.maximum(z, 0.0)
    if emit_g:
        g0_ref, g1_ref = out_refs
        g = h * ns_ref[...]
        g0_ref[...] = g[:, :DH]
        g1_ref[...] = g[:, DH:]
    else:
        (h_ref,) = out_refs
        h_ref[...] = h


def _make_layer(emit_g):
    if emit_g:
        out_specs = [pl.BlockSpec((_RB, DH), lambda i: (i, 0)),
                     pl.BlockSpec((_RB, DH), lambda i: (i, 0))]
        out_shape = [jax.ShapeDtypeStruct((N_NODES, DH), jnp.float32),
                     jax.ShapeDtypeStruct((N_NODES, DH), jnp.float32)]
    else:
        out_specs = [pl.BlockSpec((_RB, D), lambda i: (i, 0))]
        out_shape = [jax.ShapeDtypeStruct((N_NODES, D), jnp.float32)]
    return pl.pallas_call(
        functools.partial(_layer_body, emit_g=emit_g),
        grid=(N_NODES // _RB,),
        in_specs=[
            pl.BlockSpec((_RB, DH), lambda i: (i, 0)),
            pl.BlockSpec((_RB, DH), lambda i: (i, 0)),
            pl.BlockSpec((_RB, 1), lambda i: (i, 0)),
            pl.BlockSpec((_RB, 1), lambda i: (i, 0)),
            pl.BlockSpec((D, D), lambda i: (0, 0)),
            pl.BlockSpec((1, D), lambda i: (0, 0)),
        ],
        out_specs=out_specs,
        out_shape=out_shape,
    )


_layer_mid = _make_layer(emit_g=True)
_layer_last = _make_layer(emit_g=False)


def kernel(x, edge_index, W0, b0, W1, b1, W2, b2):
    src_d = edge_index[0].reshape(NS, NHALF, HCHUNKS, DCHUNK)
    dst_d = edge_index[1].reshape(NS, NHALF, HCHUNKS, DCHUNK)
    src = edge_index[0].reshape(NS, N_CHUNKS, 1, CHUNK)
    dst = edge_index[1].reshape(NS, N_CHUNKS, 1, CHUNK)

    dego_p, degi_p = _sc_degrees(src_d, dst_d)
    dego = dego_p[:N_NODES].reshape(N_NODES, 1)
    degi = degi_p[:N_NODES].reshape(N_NODES, 1)

    ns, nd, g0, g1 = _prelude(dego, degi, x)

    a0, a1 = _sc_agg(g0, g1, src, dst)
    g0, g1 = _layer_mid(a0[:N_NODES], a1[:N_NODES], nd, ns, W0, b0.reshape(1, D))

    a0, a1 = _sc_agg(g0, g1, src, dst)
    g0, g1 = _layer_mid(a0[:N_NODES], a1[:N_NODES], nd, ns, W1, b1.reshape(1, D))

    a0, a1 = _sc_agg(g0, g1, src, dst)
    (h,) = _layer_last(a0[:N_NODES], a1[:N_NODES], nd, ns, W2, b2.reshape(1, D))
    return h


# speedup vs baseline: 1.4189x; 1.0727x over previous
import functools

import jax
import jax.numpy as jnp
from jax import lax
from jax.experimental import pallas as pl
from jax.experimental.pallas import tpu as pltpu
from jax.experimental.pallas import tpu_sc as plsc

N_NODES = 10000
N_EDGES = 160000
D = 256
DH = 128
NC, NS, L = 2, 16, 16

EDGES_PER_TILE = N_EDGES // NS
CHUNK = 100
N_CHUNKS = EDGES_PER_TILE // CHUNK
NBUF = 3
DCHUNK = 100
NHALF = 2
HCHUNKS = (EDGES_PER_TILE // DCHUNK) // NHALF
NPAD = 10240
ROWS_PER_TILE = NPAD // NS
ZROWS = 80

HIST_PAD = 10240
HROWS = HIST_PAD // NS


def _zero_vec_ref(ref, n):
    def body(i, _):
        ref[pl.ds(i * 16, 16)] = jnp.zeros((16,), jnp.float32)
        return 0
    lax.fori_loop(0, n // 16, body, 0)


def _zero_mat_ref(ref, rows, cols):
    def body(i, _):
        def inner(j, _):
            ref[i, pl.ds(j * 16, 16)] = jnp.zeros((16,), jnp.float32)
            return 0
        lax.fori_loop(0, cols // 16, inner, 0)
        return 0
    lax.fori_loop(0, rows, body, 0)


def _sc_degrees_body(src_hbm, dst_hbm, dego_hbm, degi_hbm,
                     idx_all, ones_v, bounce_v, acc_sh):
    c = lax.axis_index("c")
    s = lax.axis_index("s")

    _zero_vec_ref(bounce_v, HROWS)

    def ones_body(i, _):
        ones_v[pl.ds(i * 16, 16)] = jnp.ones((16,), jnp.float32)
        return 0
    lax.fori_loop(0, 8, ones_body, 0)

    pltpu.sync_copy(bounce_v, acc_sh.at[pl.ds(s * HROWS, HROWS)])
    plsc.subcore_barrier()

    def run(edge_hbm):
        for hh in range(NHALF):
            pltpu.sync_copy(edge_hbm.at[s, hh], idx_all)

            def body(i, _):
                pltpu.sync_copy(ones_v.at[pl.ds(0, DCHUNK)],
                                acc_sh.at[idx_all.at[i]], add=True)
                return 0
            lax.fori_loop(0, HCHUNKS, body, 0)

    @pl.when(c == 0)
    def _():
        run(src_hbm)

    @pl.when(c == 1)
    def _():
        run(dst_hbm)

    plsc.subcore_barrier()

    @pl.when(c == 0)
    def _():
        pltpu.sync_copy(acc_sh.at[pl.ds(s * HROWS, HROWS)],
                        dego_hbm.at[pl.ds(s * HROWS, HROWS)])

    @pl.when(c == 1)
    def _():
        pltpu.sync_copy(acc_sh.at[pl.ds(s * HROWS, HROWS)],
                        degi_hbm.at[pl.ds(s * HROWS, HROWS)])


_sc_degrees = pl.kernel(
    _sc_degrees_body,
    out_type=[jax.ShapeDtypeStruct((HIST_PAD,), jnp.float32),
              jax.ShapeDtypeStruct((HIST_PAD,), jnp.float32)],
    mesh=plsc.VectorSubcoreMesh(core_axis_name="c", subcore_axis_name="s"),
    scratch_types=[
        pltpu.VMEM((HCHUNKS, DCHUNK), jnp.int32),
        pltpu.VMEM((128,), jnp.float32),
        pltpu.VMEM((HROWS,), jnp.float32),
        pltpu.VMEM_SHARED((HIST_PAD,), jnp.float32),
    ],
)


def _sc_agg_body(g0_hbm, g1_hbm, src_hbm, dst_hbm, out0_hbm, out1_hbm,
                 sidx0, sidx1, sidx2, didx0, didx1, didx2,
                 rows, acc_sh, zsem,
                 gsem0, gsem1, gsem2, isem0, isem1, isem2):
    c = lax.axis_index("c")
    s = lax.axis_index("s")
    sidx = (sidx0, sidx1, sidx2)
    didx = (didx0, didx1, didx2)
    gsems = (gsem0, gsem1, gsem2)
    isems = (isem0, isem1, isem2)

    zbuf = rows.at[0]
    _zero_mat_ref(zbuf, ZROWS, DH)
    for k in range(ROWS_PER_TILE // ZROWS):
        pltpu.async_copy(zbuf.at[pl.ds(0, ZROWS)],
                         acc_sh.at[pl.ds(s * ROWS_PER_TILE + k * ZROWS, ZROWS)], zsem)
    for k in range(NBUF - 1):
        pltpu.sync_copy(src_hbm.at[s, k], sidx[k])
        pltpu.sync_copy(dst_hbm.at[s, k], didx[k])
    for k in range(ROWS_PER_TILE // ZROWS):
        pltpu.make_async_copy(zbuf.at[pl.ds(0, ZROWS)],
                              acc_sh.at[pl.ds(s * ROWS_PER_TILE + k * ZROWS, ZROWS)], zsem).wait()
    plsc.subcore_barrier()

    def run(g_hbm):
        for k in range(NBUF - 1):
            pltpu.async_copy(g_hbm.at[sidx[k].at[0]], rows.at[k], gsems[k])
        pltpu.async_copy(src_hbm.at[s, NBUF - 1], sidx[NBUF - 1], isems[NBUF - 1])
        pltpu.async_copy(dst_hbm.at[s, NBUF - 1], didx[NBUF - 1], isems[NBUF - 1])

        def visit(cc, k):
            kn = (k + NBUF - 1) % NBUF
            pltpu.make_async_copy(g_hbm.at[sidx[k].at[0]], rows.at[k], gsems[k]).wait()

            @pl.when(cc + NBUF - 1 < N_CHUNKS)
            def _():
                pltpu.make_async_copy(src_hbm.at[s, cc + NBUF - 1], sidx[kn], isems[kn]).wait()
                pltpu.make_async_copy(dst_hbm.at[s, cc + NBUF - 1], didx[kn], isems[kn]).wait()
                pltpu.async_copy(g_hbm.at[sidx[kn].at[0]], rows.at[kn], gsems[kn])

            pltpu.sync_copy(rows.at[k], acc_sh.at[didx[k].at[0]], add=True)

            @pl.when(cc + NBUF < N_CHUNKS)
            def _():
                pltpu.async_copy(src_hbm.at[s, cc + NBUF], sidx[k], isems[k])
                pltpu.async_copy(dst_hbm.at[s, cc + NBUF], didx[k], isems[k])

        def body(j, _):
            for k in range(NBUF):
                cc = NBUF * j + k

                @pl.when(cc < N_CHUNKS)
                def _():
                    visit(cc, k)
            return 0
        lax.fori_loop(0, (N_CHUNKS + NBUF - 1) // NBUF, body, 0)

    @pl.when(c == 0)
    def _():
        run(g0_hbm)

    @pl.when(c == 1)
    def _():
        run(g1_hbm)

    plsc.subcore_barrier()

    def writeback(out_hbm):
        r0 = s * ROWS_PER_TILE
        pltpu.sync_copy(acc_sh.at[pl.ds(r0, ROWS_PER_TILE)],
                        out_hbm.at[pl.ds(r0, ROWS_PER_TILE)])

    @pl.when(c == 0)
    def _():
        writeback(out0_hbm)

    @pl.when(c == 1)
    def _():
        writeback(out1_hbm)


_sc_agg = pl.kernel(
    _sc_agg_body,
    out_type=[jax.ShapeDtypeStruct((NPAD, DH), jnp.float32),
              jax.ShapeDtypeStruct((NPAD, DH), jnp.float32)],
    mesh=plsc.VectorSubcoreMesh(core_axis_name="c", subcore_axis_name="s"),
    scratch_types=(
        [pltpu.VMEM((1, CHUNK), jnp.int32) for _ in range(2 * NBUF)]
        + [
            pltpu.VMEM((NBUF, CHUNK, DH), jnp.float32),
            pltpu.VMEM_SHARED((NPAD, DH), jnp.float32),
        ]
        + [pltpu.SemaphoreType.DMA for _ in range(1 + 2 * NBUF)]
    ),
)


_RB = 1024


def _prelude_body(dego_ref, degi_ref, x_ref, ns_ref, nd_ref, g0_ref, g1_ref):
    ns = lax.rsqrt(jnp.maximum(dego_ref[...], 1.0))
    nd = lax.rsqrt(jnp.maximum(degi_ref[...], 1.0))
    ns_ref[...] = ns
    nd_ref[...] = nd
    g = x_ref[...] * ns[:, None]
    g0_ref[...] = g[:, :DH]
    g1_ref[...] = g[:, DH:]


_prelude = pl.pallas_call(
    _prelude_body,
    grid=(NPAD // _RB,),
    in_specs=[
        pl.BlockSpec((_RB,), lambda i: (i,)),
        pl.BlockSpec((_RB,), lambda i: (i,)),
        pl.BlockSpec((_RB, D), lambda i: (i, 0)),
    ],
    out_specs=[
        pl.BlockSpec((_RB,), lambda i: (i,)),
        pl.BlockSpec((_RB,), lambda i: (i,)),
        pl.BlockSpec((_RB, DH), lambda i: (i, 0)),
        pl.BlockSpec((_RB, DH), lambda i: (i, 0)),
    ],
    out_shape=[
        jax.ShapeDtypeStruct((NPAD,), jnp.float32),
        jax.ShapeDtypeStruct((NPAD,), jnp.float32),
        jax.ShapeDtypeStruct((NPAD, DH), jnp.float32),
        jax.ShapeDtypeStruct((NPAD, DH), jnp.float32),
    ],
)


def _layer_body(a0_ref, a1_ref, nd_ref, ns_ref, w_ref, b_ref, *out_refs,
                emit_g):
    nd = nd_ref[...][:, None]
    agg = jnp.concatenate([a0_ref[...] * nd, a1_ref[...] * nd], axis=1)
    z = jnp.dot(agg, w_ref[...], preferred_element_type=jnp.float32,
                precision=lax.Precision.HIGHEST) + b_ref[...]
    h = jnp.maximum(z, 0.0)
    if emit_g:
        g0_ref, g1_ref = out_refs
        g = h * ns_ref[...][:, None]
        g0_ref[...] = g[:, :DH]
        g1_ref[...] = g[:, DH:]
    else:
        (h_ref,) = out_refs
        h_ref[...] = h


def _make_layer(emit_g):
    if emit_g:
        out_specs = [pl.BlockSpec((_RB, DH), lambda i: (i, 0)),
                     pl.BlockSpec((_RB, DH), lambda i: (i, 0))]
        out_shape = [jax.ShapeDtypeStruct((NPAD, DH), jnp.float32),
                     jax.ShapeDtypeStruct((NPAD, DH), jnp.float32)]
    else:
        out_specs = [pl.BlockSpec((_RB, D), lambda i: (i, 0))]
        out_shape = [jax.ShapeDtypeStruct((NPAD, D), jnp.float32)]
    return pl.pallas_call(
        functools.partial(_layer_body, emit_g=emit_g),
        grid=(NPAD // _RB,),
        in_specs=[
            pl.BlockSpec((_RB, DH), lambda i: (i, 0)),
            pl.BlockSpec((_RB, DH), lambda i: (i, 0)),
            pl.BlockSpec((_RB,), lambda i: (i,)),
            pl.BlockSpec((_RB,), lambda i: (i,)),
            pl.BlockSpec((D, D), lambda i: (0, 0)),
            pl.BlockSpec((1, D), lambda i: (0, 0)),
        ],
        out_specs=out_specs,
        out_shape=out_shape,
    )


_layer_mid = _make_layer(emit_g=True)
_layer_last = _make_layer(emit_g=False)


def kernel(x, edge_index, W0, b0, W1, b1, W2, b2):
    src_d = edge_index[0].reshape(NS, NHALF, HCHUNKS, DCHUNK)
    dst_d = edge_index[1].reshape(NS, NHALF, HCHUNKS, DCHUNK)
    src = edge_index[0].reshape(NS, N_CHUNKS, 1, CHUNK)
    dst = edge_index[1].reshape(NS, N_CHUNKS, 1, CHUNK)

    dego_p, degi_p = _sc_degrees(src_d, dst_d)

    x_pad = jnp.concatenate(
        [x, jnp.zeros((NPAD - N_NODES, D), jnp.float32)], axis=0)
    ns, nd, g0, g1 = _prelude(dego_p, degi_p, x_pad)

    a0, a1 = _sc_agg(g0, g1, src, dst)
    g0, g1 = _layer_mid(a0, a1, nd, ns, W0, b0.reshape(1, D))

    a0, a1 = _sc_agg(g0, g1, src, dst)
    g0, g1 = _layer_mid(a0, a1, nd, ns, W1, b1.reshape(1, D))

    a0, a1 = _sc_agg(g0, g1, src, dst)
    (h,) = _layer_last(a0, a1, nd, ns, W2, b2.reshape(1, D))
    return h[:N_NODES]
